# Initial kernel scaffold; baseline (speedup 1.0000x reference)
import jax
import jax.numpy as jnp
from jax.experimental import pallas as pl


def kernel(node, edge_index, edge_attr, batch_ptr, W0, b0, g0, be0, W1, b1, g1, be1, W2, b2, g2, be2, Wjk, bjk):
    raise NotImplementedError("write your pallas kernel here")


# speedup vs baseline: 10.3569x; 10.3569x over previous
import functools

import jax
import jax.numpy as jnp
from jax import lax
from jax.experimental import pallas as pl
from jax.experimental.pallas import tpu as pltpu
from jax.experimental.pallas import tpu_sc as plsc

N = 10000
E = 320000
D = 128
NC = 2
NS = 16
NW = NC * NS
EPT = E // NW
CH = 128
NFULL = EPT // CH
TAIL = EPT - NFULL * CH
NPAD = 10240
RPT = NPAD // NS
ZR = 128
DCH = 2000
BN = 1000
GRID = N // BN

_sc_mesh = plsc.VectorSubcoreMesh(core_axis_name="c", subcore_axis_name="s")


@functools.partial(
    pl.kernel,
    out_type=jax.ShapeDtypeStruct((NW, N), jnp.float32),
    mesh=_sc_mesh,
    compiler_params=pltpu.CompilerParams(needs_layout_passes=False),
    scratch_types=[
        pltpu.VMEM((N,), jnp.float32),
        pltpu.VMEM((DCH,), jnp.int32),
        pltpu.VMEM((DCH,), jnp.float32),
    ],
)
def _deg_kernel(col_hbm, w_hbm, out_hbm, acc, colbuf, wbuf):
    cid = lax.axis_index("c")
    sid = lax.axis_index("s")
    wid = cid * NS + sid

    def zero_body(i, _):
        acc[pl.ds(i * 16, 16)] = jnp.zeros((16,), jnp.float32)
        return 0

    lax.fori_loop(0, N // 16, zero_body, 0)

    def chunk_body(i, _):
        base = wid * EPT + i * DCH
        pltpu.sync_copy(col_hbm.at[pl.ds(base, DCH)], colbuf)
        pltpu.sync_copy(w_hbm.at[pl.ds(base, DCH)], wbuf)

        def grp(g, _):
            idx = colbuf[pl.ds(g * 16, 16)]
            val = wbuf[pl.ds(g * 16, 16)]
            plsc.addupdate_scatter(acc, [idx], val)
            return 0

        lax.fori_loop(0, DCH // 16, grp, 0)
        return 0

    lax.fori_loop(0, EPT // DCH, chunk_body, 0)
    pltpu.sync_copy(acc, out_hbm.at[wid])


@functools.partial(
    pl.kernel,
    out_type=jax.ShapeDtypeStruct((NC, NPAD, D), jnp.float32),
    mesh=_sc_mesh,
    compiler_params=pltpu.CompilerParams(needs_layout_passes=False),
    scratch_types=[
        pltpu.VMEM_SHARED((NPAD, D), jnp.float32),
        pltpu.VMEM((ZR, D), jnp.float32),
        pltpu.VMEM((1, CH), jnp.int32),
        pltpu.VMEM((1, CH), jnp.int32),
        pltpu.VMEM((1, CH), jnp.float32),
        pltpu.VMEM((CH, D), jnp.float32),
        pltpu.VMEM((1, TAIL), jnp.int32),
        pltpu.VMEM((1, TAIL), jnp.int32),
        pltpu.VMEM((1, TAIL), jnp.float32),
        pltpu.VMEM((TAIL, D), jnp.float32),
        pltpu.SemaphoreType.DMA,
    ],
)
def _agg_kernel(hp_hbm, row_hbm, col_hbm, w_hbm, out_hbm,
                acc_sh, zbuf, rowbuf, colbuf, wbuf, databuf,
                trow, tcol, tw, tbuf, gsem):
    cid = lax.axis_index("c")
    sid = lax.axis_index("s")
    wid = cid * NS + sid

    def zb(r, _):
        for f in range(D // 16):
            zbuf[r, pl.ds(f * 16, 16)] = jnp.zeros((16,), jnp.float32)
        return 0

    lax.fori_loop(0, ZR, zb, 0)
    for k in range(RPT // ZR):
        pltpu.sync_copy(zbuf, acc_sh.at[pl.ds(sid * RPT + k * ZR, ZR)])
    plsc.subcore_barrier()

    def scale_rows(buf, wref, nrows):
        def grp(g, _):
            w16 = wref[0, pl.ds(g * 16, 16)]
            for e in range(16):
                w_s = w16[e]
                for f in range(D // 16):
                    v = buf[g * 16 + e, pl.ds(f * 16, 16)]
                    buf[g * 16 + e, pl.ds(f * 16, 16)] = v * w_s
            return 0

        lax.fori_loop(0, nrows // 16, grp, 0)

    def chunk(i, _):
        base = wid * EPT + i * CH
        pltpu.sync_copy(row_hbm.at[pl.ds(base, CH)], rowbuf.at[0])
        pltpu.sync_copy(col_hbm.at[pl.ds(base, CH)], colbuf.at[0])
        pltpu.sync_copy(w_hbm.at[pl.ds(base, CH)], wbuf.at[0])
        pltpu.async_copy(hp_hbm.at[rowbuf.at[0]], databuf, gsem).wait()
        scale_rows(databuf, wbuf, CH)
        pltpu.sync_copy(databuf, acc_sh.at[colbuf.at[0]], add=True)
        return 0

    lax.fori_loop(0, NFULL, chunk, 0)

    tbase = wid * EPT + NFULL * CH
    pltpu.sync_copy(row_hbm.at[pl.ds(tbase, TAIL)], trow.at[0])
    pltpu.sync_copy(col_hbm.at[pl.ds(tbase, TAIL)], tcol.at[0])
    pltpu.sync_copy(w_hbm.at[pl.ds(tbase, TAIL)], tw.at[0])
    pltpu.async_copy(hp_hbm.at[trow.at[0]], tbuf, gsem).wait()
    scale_rows(tbuf, tw, TAIL)
    pltpu.sync_copy(tbuf, acc_sh.at[tcol.at[0]], add=True)

    plsc.subcore_barrier()
    pltpu.sync_copy(acc_sh.at[pl.ds(sid * RPT, RPT)],
                    out_hbm.at[cid, pl.ds(sid * RPT, RPT)])


def _dis_from(degp):
    deg = jnp.sum(degp, axis=-1) + 1.0
    return jnp.where(deg > 0, lax.rsqrt(deg), 0.0)


def _tc_prep_body(degp_ref, x_ref, w0_ref, hp_ref):
    dis = _dis_from(degp_ref[...])
    hp_ref[...] = jnp.dot(x_ref[...], w0_ref[...],
                          preferred_element_type=jnp.float32) * dis[:, None]


def _ln_relu(y, g, be):
    mu = jnp.mean(y, axis=-1, keepdims=True)
    var = jnp.mean((y - mu) ** 2, axis=-1, keepdims=True)
    return jnp.maximum((y - mu) * lax.rsqrt(var + 1e-5) * g + be, 0.0)


def _tc_post_body(degp_ref, x_ref, hp_ref, acc_ref, b_ref, g_ref, be_ref,
                  wn_ref, wjk_ref, jk_ref, xn_ref, hpn_ref, jko_ref,
                  *, first):
    dis = _dis_from(degp_ref[...])
    o = dis[:, None] * (acc_ref[0] + acc_ref[1] + hp_ref[...]) + b_ref[...]
    xn = _ln_relu(x_ref[...] + o, g_ref[...], be_ref[...])
    xn_ref[...] = xn
    hpn_ref[...] = jnp.dot(xn, wn_ref[...],
                           preferred_element_type=jnp.float32) * dis[:, None]
    jk = jnp.dot(xn, wjk_ref[...], preferred_element_type=jnp.float32)
    if not first:
        jk = jk + jk_ref[...]
    jko_ref[...] = jk


def _tc_final_body(degp_ref, x_ref, hp_ref, acc_ref, b_ref, g_ref, be_ref,
                   wjk_ref, bjk_ref, jk_ref, out_ref):
    dis = _dis_from(degp_ref[...])
    o = dis[:, None] * (acc_ref[0] + acc_ref[1] + hp_ref[...]) + b_ref[...]
    xn = _ln_relu(x_ref[...] + o, g_ref[...], be_ref[...])
    out_ref[...] = (jk_ref[...] + bjk_ref[...]
                    + jnp.dot(xn, wjk_ref[...],
                              preferred_element_type=jnp.float32))


_b_degp = pl.BlockSpec((BN, NW), lambda i: (i, 0))
_b_rows = pl.BlockSpec((BN, D), lambda i: (i, 0))
_b_acc = pl.BlockSpec((NC, BN, D), lambda i: (0, i, 0))
_b_w = pl.BlockSpec((D, D), lambda i: (0, 0))
_b_vec = pl.BlockSpec((1, D), lambda i: (0, 0))

_f32 = jnp.float32
_nd = jax.ShapeDtypeStruct((N, D), _f32)

_tc_prep = pl.pallas_call(
    _tc_prep_body,
    grid=(GRID,),
    in_specs=[_b_degp, _b_rows, _b_w],
    out_specs=_b_rows,
    out_shape=_nd,
)


def _make_post(first):
    return pl.pallas_call(
        functools.partial(_tc_post_body, first=first),
        grid=(GRID,),
        in_specs=[_b_degp, _b_rows, _b_rows, _b_acc, _b_vec, _b_vec, _b_vec,
                  _b_w, _b_w, _b_rows],
        out_specs=[_b_rows, _b_rows, _b_rows],
        out_shape=[_nd, _nd, _nd],
    )


_tc_post0 = _make_post(True)
_tc_post1 = _make_post(False)

_tc_final = pl.pallas_call(
    _tc_final_body,
    grid=(GRID,),
    in_specs=[_b_degp, _b_rows, _b_rows, _b_acc, _b_vec, _b_vec, _b_vec,
              _b_w, _b_vec, _b_rows],
    out_specs=_b_rows,
    out_shape=_nd,
)


def kernel(node, edge_index, edge_attr, batch_ptr,
           W0, b0, g0, be0, W1, b1, g1, be1, W2, b2, g2, be2,
           Wjk, bjk):
    del batch_ptr
    row = edge_index[0].astype(jnp.int32)
    col = edge_index[1].astype(jnp.int32)
    w = edge_attr.astype(jnp.float32)

    degp = _deg_kernel(col, w).T

    b0r, g0r, be0r = b0.reshape(1, D), g0.reshape(1, D), be0.reshape(1, D)
    b1r, g1r, be1r = b1.reshape(1, D), g1.reshape(1, D), be1.reshape(1, D)
    b2r, g2r, be2r = b2.reshape(1, D), g2.reshape(1, D), be2.reshape(1, D)
    wjk0, wjk1, wjk2 = Wjk[:D], Wjk[D:2 * D], Wjk[2 * D:]
    bjkr = bjk.reshape(1, D)

    hp0 = _tc_prep(degp, node, W0)
    acc0 = _agg_kernel(hp0, row, col, w)
    x1, hp1, jk = _tc_post0(degp, node, hp0, acc0, b0r, g0r, be0r,
                            W1, wjk0, jnp.zeros((N, D), _f32))
    acc1 = _agg_kernel(hp1, row, col, w)
    x2, hp2, jk = _tc_post1(degp, x1, hp1, acc1, b1r, g1r, be1r,
                            W2, wjk1, jk)
    acc2 = _agg_kernel(hp2, row, col, w)
    return _tc_final(degp, x2, hp2, acc2, b2r, g2r, be2r, wjk2, bjkr, jk)
